# baseline (device time: 32416 ns/iter reference)
import jax
import jax.numpy as jnp
from jax import lax
from jax.experimental import pallas as pl
from jax.experimental.pallas import tpu as pltpu

N_DEV = 4
N_PHASES = 6


def kernel(x, Win0, Wout0, Win1, Wout1, Win2, Wout2):
    b_per, d = x.shape

    def body(
        x_ref, win0_ref, wout0_ref, win1_ref, wout1_ref, win2_ref, wout2_ref,
        out_ref, xfull_ref, partial_ref, xloc_ref, rs_recv_ref,
        send_sems, recv_sems,
    ):
        my = lax.axis_index("i")

        barrier_sem = pltpu.get_barrier_semaphore()
        for dd in range(1, N_DEV):
            peer = lax.rem(my + dd, N_DEV)
            pl.semaphore_signal(
                barrier_sem, inc=1,
                device_id=(peer,), device_id_type=pl.DeviceIdType.MESH,
            )
        pl.semaphore_wait(barrier_sem, N_DEV - 1)

        def ag_phase(p, src_ref):
            xfull_ref[pl.ds(my * b_per, b_per), :] = src_ref[...]
            sends = []
            for dd in range(1, N_DEV):
                tgt = lax.rem(my + dd, N_DEV)
                rdma = pltpu.make_async_remote_copy(
                    src_ref=src_ref,
                    dst_ref=xfull_ref.at[pl.ds(my * b_per, b_per), :],
                    send_sem=send_sems.at[p, dd - 1],
                    recv_sem=recv_sems.at[p, dd - 1],
                    device_id=(tgt,),
                    device_id_type=pl.DeviceIdType.MESH,
                )
                rdma.start()
                sends.append(rdma)
            for dd in range(1, N_DEV):
                src_peer = lax.rem(my - dd + N_DEV, N_DEV)
                recv = pltpu.make_async_remote_copy(
                    src_ref=xloc_ref,
                    dst_ref=xfull_ref.at[pl.ds(src_peer * b_per, b_per), :],
                    send_sem=send_sems.at[p, dd - 1],
                    recv_sem=recv_sems.at[p, dd - 1],
                    device_id=(my,),
                    device_id_type=pl.DeviceIdType.MESH,
                )
                recv.wait_recv()
            for rdma in sends:
                rdma.wait_send()

        def rs_phase(p, dst_local_ref):
            sends = []
            for dd in range(1, N_DEV):
                tgt = lax.rem(my + dd, N_DEV)
                rdma = pltpu.make_async_remote_copy(
                    src_ref=partial_ref.at[pl.ds(tgt * b_per, b_per), :],
                    dst_ref=rs_recv_ref.at[dd - 1],
                    send_sem=send_sems.at[p, dd - 1],
                    recv_sem=recv_sems.at[p, dd - 1],
                    device_id=(tgt,),
                    device_id_type=pl.DeviceIdType.MESH,
                )
                rdma.start()
                sends.append(rdma)
            for dd in range(1, N_DEV):
                recv = pltpu.make_async_remote_copy(
                    src_ref=xloc_ref,
                    dst_ref=rs_recv_ref.at[dd - 1],
                    send_sem=send_sems.at[p, dd - 1],
                    recv_sem=recv_sems.at[p, dd - 1],
                    device_id=(my,),
                    device_id_type=pl.DeviceIdType.MESH,
                )
                recv.wait_recv()
            acc = (
                partial_ref[pl.ds(my * b_per, b_per), :]
                + rs_recv_ref[0]
                + rs_recv_ref[1]
                + rs_recv_ref[2]
            )
            dst_local_ref[...] = acc
            for rdma in sends:
                rdma.wait_send()

        def layer(win_ref, wout_ref):
            xb = xfull_ref[...].astype(jnp.bfloat16)
            w1 = win_ref[...].astype(jnp.bfloat16)
            h = jnp.dot(xb, w1, preferred_element_type=jnp.float32)
            h = jnp.maximum(h, 0.0).astype(jnp.bfloat16)
            w2 = wout_ref[...].astype(jnp.bfloat16)
            partial_ref[...] = jnp.dot(h, w2, preferred_element_type=jnp.float32)

        ag_phase(0, x_ref)
        layer(win0_ref, wout0_ref)
        rs_phase(1, xloc_ref)

        ag_phase(2, xloc_ref)
        layer(win1_ref, wout1_ref)
        rs_phase(3, xloc_ref)

        ag_phase(4, xloc_ref)
        layer(win2_ref, wout2_ref)
        rs_phase(5, out_ref)

    return pl.pallas_call(
        body,
        out_shape=jax.ShapeDtypeStruct((b_per, d), jnp.float32),
        in_specs=[pl.BlockSpec(memory_space=pltpu.VMEM)] * 7,
        out_specs=pl.BlockSpec(memory_space=pltpu.VMEM),
        scratch_shapes=[
            pltpu.VMEM((N_DEV * b_per, d), jnp.float32),
            pltpu.VMEM((N_DEV * b_per, d), jnp.float32),
            pltpu.VMEM((b_per, d), jnp.float32),
            pltpu.VMEM((N_DEV - 1, b_per, d), jnp.float32),
            pltpu.SemaphoreType.DMA((N_PHASES, N_DEV - 1)),
            pltpu.SemaphoreType.DMA((N_PHASES, N_DEV - 1)),
        ],
        compiler_params=pltpu.CompilerParams(collective_id=0),
    )(x, Win0, Wout0, Win1, Wout1, Win2, Wout2)


# device time: 20874 ns/iter; 1.5529x vs baseline; 1.5529x over previous
import jax
import jax.numpy as jnp
from jax import lax
from jax.experimental import pallas as pl
from jax.experimental.pallas import tpu as pltpu

N_DEV = 4
N_LAYERS = 3
H_PER = 256


def kernel(x, Win0, Wout0, Win1, Wout1, Win2, Wout2):
    b_per, d = x.shape

    def body(
        x_ref, win0_ref, wout0_ref, win1_ref, wout1_ref, win2_ref, wout2_ref,
        out_ref, win_full_ref, wout_full_ref,
        send_sems, recv_sems,
    ):
        my = lax.axis_index("i")
        win_refs = [win0_ref, win1_ref, win2_ref]
        wout_refs = [wout0_ref, wout1_ref, wout2_ref]

        for l in range(N_LAYERS):
            win_full_ref[l, :, pl.ds(0, H_PER)] = win_refs[l][...].astype(
                jnp.bfloat16
            )
            wout_full_ref[l, pl.ds(0, H_PER), :] = wout_refs[l][...].astype(
                jnp.bfloat16
            )

        barrier_sem = pltpu.get_barrier_semaphore()
        for dd in range(1, N_DEV):
            peer = lax.rem(my + dd, N_DEV)
            pl.semaphore_signal(
                barrier_sem, inc=1,
                device_id=(peer,), device_id_type=pl.DeviceIdType.MESH,
            )
        pl.semaphore_wait(barrier_sem, N_DEV - 1)

        sends = []
        for l in range(N_LAYERS):
            for dd in range(1, N_DEV):
                tgt = lax.rem(my + dd, N_DEV)
                w_rdma = pltpu.make_async_remote_copy(
                    src_ref=win_full_ref.at[l, :, pl.ds(0, H_PER)],
                    dst_ref=win_full_ref.at[l, :, pl.ds(dd * H_PER, H_PER)],
                    send_sem=send_sems.at[0, l, dd - 1],
                    recv_sem=recv_sems.at[0, l, dd - 1],
                    device_id=(tgt,),
                    device_id_type=pl.DeviceIdType.MESH,
                )
                w_rdma.start()
                sends.append(w_rdma)
                o_rdma = pltpu.make_async_remote_copy(
                    src_ref=wout_full_ref.at[l, pl.ds(0, H_PER), :],
                    dst_ref=wout_full_ref.at[l, pl.ds(dd * H_PER, H_PER), :],
                    send_sem=send_sems.at[1, l, dd - 1],
                    recv_sem=recv_sems.at[1, l, dd - 1],
                    device_id=(tgt,),
                    device_id_type=pl.DeviceIdType.MESH,
                )
                o_rdma.start()
                sends.append(o_rdma)

        def wait_recv(kind, l, dd):
            if kind == 0:
                dst = win_full_ref.at[l, :, pl.ds(dd * H_PER, H_PER)]
                src = win_full_ref.at[l, :, pl.ds(0, H_PER)]
            else:
                dst = wout_full_ref.at[l, pl.ds(dd * H_PER, H_PER), :]
                src = wout_full_ref.at[l, pl.ds(0, H_PER), :]
            pltpu.make_async_remote_copy(
                src_ref=src,
                dst_ref=dst,
                send_sem=send_sems.at[kind, l, dd - 1],
                recv_sem=recv_sems.at[kind, l, dd - 1],
                device_id=(my,),
                device_id_type=pl.DeviceIdType.MESH,
            ).wait_recv()

        xb = x_ref[...].astype(jnp.bfloat16)
        xf = None
        for l in range(N_LAYERS):
            for dd in range(1, N_DEV):
                wait_recv(0, l, dd)
            h = jnp.dot(
                xb, win_full_ref[l], preferred_element_type=jnp.float32
            )
            hb = jnp.maximum(h, 0.0).astype(jnp.bfloat16)
            for dd in range(1, N_DEV):
                wait_recv(1, l, dd)
            xf = jnp.dot(
                hb, wout_full_ref[l], preferred_element_type=jnp.float32
            )
            xb = xf.astype(jnp.bfloat16)
        out_ref[...] = xf

        for rdma in sends:
            rdma.wait_send()

    return pl.pallas_call(
        body,
        out_shape=jax.ShapeDtypeStruct((b_per, d), jnp.float32),
        in_specs=[pl.BlockSpec(memory_space=pltpu.VMEM)] * 7,
        out_specs=pl.BlockSpec(memory_space=pltpu.VMEM),
        scratch_shapes=[
            pltpu.VMEM((N_LAYERS, d, N_DEV * H_PER), jnp.bfloat16),
            pltpu.VMEM((N_LAYERS, N_DEV * H_PER, d), jnp.bfloat16),
            pltpu.SemaphoreType.DMA((2, N_LAYERS, N_DEV - 1)),
            pltpu.SemaphoreType.DMA((2, N_LAYERS, N_DEV - 1)),
        ],
        compiler_params=pltpu.CompilerParams(collective_id=0),
    )(x, Win0, Wout0, Win1, Wout1, Win2, Wout2)
